# Initial kernel scaffold; baseline (speedup 1.0000x reference)
import jax
import jax.numpy as jnp
from jax import lax
from jax.experimental import pallas as pl
from jax.experimental.pallas import tpu as pltpu
from jax.experimental.pallas import tpu_sc as plsc

N = 10000
NP = 10240
D = 128
E = 320000
EP = 327680
EW = EP // 32
CHUNK = 128
DCHUNK = 2048
BR = 2048

_mesh = plsc.VectorSubcoreMesh(core_axis_name="c", subcore_axis_name="s")


def _deg_body(dst_hbm, out_hbm, dbuf, cnt):
    c = lax.axis_index("c")
    s = lax.axis_index("s")
    wid = s * 2 + c

    def zero(j, _):
        cnt[pl.ds(j * 16, 16)] = jnp.zeros((16,), jnp.float32)
        return ()
    lax.fori_loop(0, NP // 16, zero, ())

    ones = jnp.full((16,), 1.0, jnp.float32)
    base = wid * EW

    def outer(k, _):
        pltpu.sync_copy(dst_hbm.at[pl.ds(base + k * DCHUNK, DCHUNK)], dbuf)

        def inner(i, _):
            idx = dbuf[pl.ds(i * 16, 16)]
            plsc.addupdate_scatter(cnt, [idx], ones)
            return ()
        lax.fori_loop(0, DCHUNK // 16, inner, ())
        return ()
    lax.fori_loop(0, EW // DCHUNK, outer, ())

    pltpu.sync_copy(cnt, out_hbm.at[wid])


_deg_call = pl.kernel(
    _deg_body,
    out_type=jax.ShapeDtypeStruct((32, NP), jnp.float32),
    mesh=_mesh,
    scratch_types=[
        pltpu.VMEM((DCHUNK,), jnp.int32),
        pltpu.VMEM((NP,), jnp.float32),
    ],
)


def _agg_body(g_hbm, src_hbm, dst_hbm, z_hbm, out_hbm, sidx, didx, rows, acc, sem):
    c = lax.axis_index("c")
    s = lax.axis_index("s")
    wid = s * 2 + c
    stripe = NP // 16

    pltpu.sync_copy(z_hbm.at[pl.ds(s * stripe, stripe), :],
                    acc.at[pl.ds(s * stripe, stripe), :])
    plsc.subcore_barrier()

    base = wid * EW

    def body(k, _):
        off = base + k * CHUNK
        pltpu.sync_copy(src_hbm.at[pl.ds(off, CHUNK)], sidx)
        pltpu.sync_copy(dst_hbm.at[pl.ds(off, CHUNK)], didx)
        pltpu.async_copy(g_hbm.at[sidx], rows, sem).wait()
        pltpu.sync_copy(rows, acc.at[didx], add=True)
        return ()
    lax.fori_loop(0, EW // CHUNK, body, ())

    plsc.subcore_barrier()
    pltpu.sync_copy(acc.at[pl.ds(s * stripe, stripe), :],
                    out_hbm.at[c, pl.ds(s * stripe, stripe), :])


_agg_call = pl.kernel(
    _agg_body,
    out_type=jax.ShapeDtypeStruct((2, NP, D), jnp.float32),
    mesh=_mesh,
    scratch_types=[
        pltpu.VMEM((CHUNK,), jnp.int32),
        pltpu.VMEM((CHUNK,), jnp.int32),
        pltpu.VMEM((CHUNK, D), jnp.float32),
        pltpu.VMEM_SHARED((NP, D), jnp.float32),
        pltpu.SemaphoreType.DMA,
    ],
)


def _dinv(cnt_block):
    deg = jnp.sum(cnt_block, axis=0) + 1.0
    return lax.rsqrt(deg)[:, None]


def _tc1_body(x_ref, w_ref, cnt_ref, g_ref):
    h = jnp.dot(x_ref[...], w_ref[...], preferred_element_type=jnp.float32)
    g_ref[...] = h * _dinv(cnt_ref[...])


def _tc2_body(agg_ref, g1_ref, cnt_ref, b_ref, w_ref, g2_ref):
    dinv = _dinv(cnt_ref[...])
    out1 = (agg_ref[0] + agg_ref[1] + g1_ref[...]) * dinv + b_ref[...]
    h2 = jnp.dot(jnp.maximum(out1, 0.0), w_ref[...],
                 preferred_element_type=jnp.float32)
    g2_ref[...] = h2 * dinv


def _tc3_body(agg_ref, g2_ref, cnt_ref, b_ref, out_ref):
    dinv = _dinv(cnt_ref[...])
    out_ref[...] = (agg_ref[0] + agg_ref[1] + g2_ref[...]) * dinv + b_ref[...]


_row_spec = pl.BlockSpec((BR, D), lambda i: (i, 0))
_cnt_spec = pl.BlockSpec((32, BR), lambda i: (0, i))
_full_spec = pl.BlockSpec((D, D), lambda i: (0, 0))
_bias_spec = pl.BlockSpec((1, D), lambda i: (0, 0))
_agg_spec = pl.BlockSpec((2, BR, D), lambda i: (0, i, 0))
_out_sds = jax.ShapeDtypeStruct((NP, D), jnp.float32)

_tc1 = pl.pallas_call(
    _tc1_body, grid=(NP // BR,),
    in_specs=[_row_spec, _full_spec, _cnt_spec],
    out_specs=_row_spec, out_shape=_out_sds)

_tc2 = pl.pallas_call(
    _tc2_body, grid=(NP // BR,),
    in_specs=[_agg_spec, _row_spec, _cnt_spec, _bias_spec, _full_spec],
    out_specs=_row_spec, out_shape=_out_sds)

_tc3 = pl.pallas_call(
    _tc3_body, grid=(NP // BR,),
    in_specs=[_agg_spec, _row_spec, _cnt_spec, _bias_spec],
    out_specs=_row_spec, out_shape=_out_sds)


@jax.jit
def kernel(x, edge_index, W1, b1, W2, b2):
    src = edge_index[0].astype(jnp.int32)
    dst = edge_index[1].astype(jnp.int32)
    pad = jnp.full((EP - E,), N, jnp.int32)
    src_p = jnp.concatenate([src, pad])
    dst_p = jnp.concatenate([dst, pad])
    x_p = jnp.pad(x, ((0, NP - N), (0, 0)))
    zeros = jnp.zeros((NP, D), jnp.float32)
    b1r = b1.reshape(1, D)
    b2r = b2.reshape(1, D)

    counts = _deg_call(dst_p)
    g1 = _tc1(x_p, W1, counts)
    agg1 = _agg_call(g1, src_p, dst_p, zeros)
    g2 = _tc2(agg1, g1, counts, b1r, W2)
    agg2 = _agg_call(g2, src_p, dst_p, zeros)
    out = _tc3(agg2, g2, counts, b2r)
    return out[:N]


# speedup vs baseline: 7.7369x; 7.7369x over previous
import jax
import jax.numpy as jnp
from jax import lax
from jax.experimental import pallas as pl
from jax.experimental.pallas import tpu as pltpu
from jax.experimental.pallas import tpu_sc as plsc

N = 10000
NP = 10240
D = 128
E = 320000
EP = 327680
EW = EP // 32
CHUNK = 128
DCHUNK = 2048
BR = 2048

_mesh = plsc.VectorSubcoreMesh(core_axis_name="c", subcore_axis_name="s")
_sc_params = pltpu.CompilerParams(needs_layout_passes=False)


def _deg_body(dst_hbm, out_hbm, dbuf, cnt):
    c = lax.axis_index("c")
    s = lax.axis_index("s")
    wid = s * 2 + c

    def zero(j, _):
        cnt[pl.ds(j * 16, 16)] = jnp.zeros((16,), jnp.float32)
        return ()
    lax.fori_loop(0, NP // 16, zero, ())

    ones = jnp.full((16,), 1.0, jnp.float32)
    base = wid * EW

    def outer(k, _):
        pltpu.sync_copy(dst_hbm.at[pl.ds(base + k * DCHUNK, DCHUNK)], dbuf)

        def inner(i, _):
            idx = dbuf[pl.ds(i * 16, 16)]
            plsc.addupdate_scatter(cnt, [idx], ones)
            return ()
        lax.fori_loop(0, DCHUNK // 16, inner, ())
        return ()
    lax.fori_loop(0, EW // DCHUNK, outer, ())

    pltpu.sync_copy(cnt, out_hbm.at[wid])


_deg_call = pl.kernel(
    _deg_body,
    out_type=jax.ShapeDtypeStruct((32, NP), jnp.float32),
    mesh=_mesh,
    scratch_types=[
        pltpu.VMEM((DCHUNK,), jnp.int32),
        pltpu.VMEM((NP,), jnp.float32),
    ],
    compiler_params=_sc_params,
)


def _agg_body(g_hbm, src_hbm, dst_hbm, z_hbm, out_hbm, sidx, didx, rows, acc, sem):
    c = lax.axis_index("c")
    s = lax.axis_index("s")
    wid = s * 2 + c
    stripe = NP // 16

    pltpu.sync_copy(z_hbm.at[pl.ds(s * stripe, stripe), :],
                    acc.at[pl.ds(s * stripe, stripe), :])
    plsc.subcore_barrier()

    base = wid * EW

    def body(k, _):
        off = base + k * CHUNK
        pltpu.sync_copy(src_hbm.at[pl.ds(off, CHUNK)], sidx)
        pltpu.sync_copy(dst_hbm.at[pl.ds(off, CHUNK)], didx)
        pltpu.async_copy(g_hbm.at[sidx], rows, sem).wait()
        pltpu.sync_copy(rows, acc.at[didx], add=True)
        return ()
    lax.fori_loop(0, EW // CHUNK, body, ())

    plsc.subcore_barrier()
    pltpu.sync_copy(acc.at[pl.ds(s * stripe, stripe), :],
                    out_hbm.at[c, pl.ds(s * stripe, stripe), :])


_agg_call = pl.kernel(
    _agg_body,
    out_type=jax.ShapeDtypeStruct((2, NP, D), jnp.float32),
    mesh=_mesh,
    scratch_types=[
        pltpu.VMEM((CHUNK,), jnp.int32),
        pltpu.VMEM((CHUNK,), jnp.int32),
        pltpu.VMEM((CHUNK, D), jnp.float32),
        pltpu.VMEM_SHARED((NP, D), jnp.float32),
        pltpu.SemaphoreType.DMA,
    ],
    compiler_params=_sc_params,
)


def _dinv(cnt_block):
    deg = jnp.sum(cnt_block, axis=0) + 1.0
    return lax.rsqrt(deg)[:, None]


def _tc1_body(x_ref, w_ref, cnt_ref, g_ref):
    h = jnp.dot(x_ref[...], w_ref[...], preferred_element_type=jnp.float32)
    g_ref[...] = h * _dinv(cnt_ref[...])


def _tc2_body(agg_ref, g1_ref, cnt_ref, b_ref, w_ref, g2_ref):
    dinv = _dinv(cnt_ref[...])
    out1 = (agg_ref[0] + agg_ref[1] + g1_ref[...]) * dinv + b_ref[...]
    h2 = jnp.dot(jnp.maximum(out1, 0.0), w_ref[...],
                 preferred_element_type=jnp.float32)
    g2_ref[...] = h2 * dinv


def _tc3_body(agg_ref, g2_ref, cnt_ref, b_ref, out_ref):
    dinv = _dinv(cnt_ref[...])
    out_ref[...] = (agg_ref[0] + agg_ref[1] + g2_ref[...]) * dinv + b_ref[...]


_row_spec = pl.BlockSpec((BR, D), lambda i: (i, 0))
_cnt_spec = pl.BlockSpec((32, BR), lambda i: (0, i))
_full_spec = pl.BlockSpec((D, D), lambda i: (0, 0))
_bias_spec = pl.BlockSpec((1, D), lambda i: (0, 0))
_agg_spec = pl.BlockSpec((2, BR, D), lambda i: (0, i, 0))
_out_sds = jax.ShapeDtypeStruct((NP, D), jnp.float32)

_tc1 = pl.pallas_call(
    _tc1_body, grid=(NP // BR,),
    in_specs=[_row_spec, _full_spec, _cnt_spec],
    out_specs=_row_spec, out_shape=_out_sds)

_tc2 = pl.pallas_call(
    _tc2_body, grid=(NP // BR,),
    in_specs=[_agg_spec, _row_spec, _cnt_spec, _bias_spec, _full_spec],
    out_specs=_row_spec, out_shape=_out_sds)

_tc3 = pl.pallas_call(
    _tc3_body, grid=(NP // BR,),
    in_specs=[_agg_spec, _row_spec, _cnt_spec, _bias_spec],
    out_specs=_row_spec, out_shape=_out_sds)


@jax.jit
def kernel(x, edge_index, W1, b1, W2, b2):
    src = edge_index[0].astype(jnp.int32)
    dst = edge_index[1].astype(jnp.int32)
    pad = jnp.full((EP - E,), N, jnp.int32)
    src_p = jnp.concatenate([src, pad])
    dst_p = jnp.concatenate([dst, pad])
    x_p = jnp.pad(x, ((0, NP - N), (0, 0)))
    zeros = jnp.zeros((NP, D), jnp.float32)
    b1r = b1.reshape(1, D)
    b2r = b2.reshape(1, D)

    counts = _deg_call(dst_p)
    g1 = _tc1(x_p, W1, counts)
    agg1 = _agg_call(g1, src_p, dst_p, zeros)
    g2 = _tc2(agg1, g1, counts, b1r, W2)
    agg2 = _agg_call(g2, src_p, dst_p, zeros)
    out = _tc3(agg2, g2, counts, b2r)
    return out[:N]


# speedup vs baseline: 8.9064x; 1.1512x over previous
import jax
import jax.numpy as jnp
from jax import lax
from jax.experimental import pallas as pl
from jax.experimental.pallas import tpu as pltpu
from jax.experimental.pallas import tpu_sc as plsc

N = 10000
NP = 10240
D = 128
E = 320000
EP = 327680
EW = EP // 32
CHUNK = 128
DCHUNK = 2048
BR = 2048

_mesh = plsc.VectorSubcoreMesh(core_axis_name="c", subcore_axis_name="s")
_sc_params = pltpu.CompilerParams(needs_layout_passes=False)


def _deg_body(dst_hbm, out_hbm, dbuf, cnt):
    c = lax.axis_index("c")
    s = lax.axis_index("s")
    wid = s * 2 + c

    def zero(j, _):
        cnt[pl.ds(j * 16, 16)] = jnp.zeros((16,), jnp.float32)
        return ()
    lax.fori_loop(0, NP // 16, zero, ())

    ones = jnp.full((16,), 1.0, jnp.float32)
    base = wid * EW

    def outer(k, _):
        pltpu.sync_copy(dst_hbm.at[pl.ds(base + k * DCHUNK, DCHUNK)], dbuf)

        def inner(i, _):
            idx = dbuf[pl.ds(i * 16, 16)]
            plsc.addupdate_scatter(cnt, [idx], ones)
            return ()
        lax.fori_loop(0, DCHUNK // 16, inner, ())
        return ()
    lax.fori_loop(0, EW // DCHUNK, outer, ())

    pltpu.sync_copy(cnt, out_hbm.at[wid])


_deg_call = pl.kernel(
    _deg_body,
    out_type=jax.ShapeDtypeStruct((32, NP), jnp.float32),
    mesh=_mesh,
    scratch_types=[
        pltpu.VMEM((DCHUNK,), jnp.int32),
        pltpu.VMEM((NP,), jnp.float32),
    ],
    compiler_params=_sc_params,
)


RW = EW // CHUNK
G = 8
NG = RW // G


def _agg_body(g_hbm, src_hbm, dst_hbm, z_hbm, out_hbm,
              sidx0, sidx1, didx0, didx1, rows0, rows1, acc,
              semr0, semr1, semi0, semi1):
    c = lax.axis_index("c")
    s = lax.axis_index("s")
    wid = s * 2 + c
    stripe = NP // 16

    sidx = (sidx0, sidx1)
    didx = (didx0, didx1)
    rows = (rows0, rows1)
    semr = (semr0, semr1)
    semi = (semi0, semi1)
    base = wid * RW

    pltpu.sync_copy(z_hbm.at[pl.ds(s * stripe, stripe), :],
                    acc.at[pl.ds(s * stripe, stripe), :])
    pltpu.async_copy(src_hbm.at[pl.ds(base, G), :], sidx0, semi0)
    pltpu.async_copy(dst_hbm.at[pl.ds(base, G), :], didx0, semi0)
    pltpu.async_copy(src_hbm.at[pl.ds(base + G, G), :], sidx1, semi1)
    pltpu.async_copy(dst_hbm.at[pl.ds(base + G, G), :], didx1, semi1)
    plsc.subcore_barrier()

    @pl.loop(0, NG, step=2)
    def _(ng2):
        for gb in range(2):
            ng = ng2 + gb
            row0 = base + ng * G
            pltpu.make_async_copy(src_hbm.at[pl.ds(row0, G), :],
                                  sidx[gb], semi[gb]).wait()
            pltpu.make_async_copy(dst_hbm.at[pl.ds(row0, G), :],
                                  didx[gb], semi[gb]).wait()
            descs = [
                pltpu.async_copy(g_hbm.at[sidx[gb].at[0]], rows0, semr0),
                pltpu.async_copy(g_hbm.at[sidx[gb].at[1]], rows1, semr1),
            ]
            for k in range(G):
                b = k % 2
                descs[b].wait()
                pltpu.sync_copy(rows[b], acc.at[didx[gb].at[k]], add=True)
                if k + 2 < G:
                    descs[b] = pltpu.async_copy(
                        g_hbm.at[sidx[gb].at[k + 2]], rows[b], semr[b])
            @pl.when(ng + 2 < NG)
            def _():
                nxt = base + (ng + 2) * G
                pltpu.async_copy(src_hbm.at[pl.ds(nxt, G), :],
                                 sidx[gb], semi[gb])
                pltpu.async_copy(dst_hbm.at[pl.ds(nxt, G), :],
                                 didx[gb], semi[gb])

    plsc.subcore_barrier()
    pltpu.sync_copy(acc.at[pl.ds(s * stripe, stripe), :],
                    out_hbm.at[c, pl.ds(s * stripe, stripe), :])


_agg_call = pl.kernel(
    _agg_body,
    out_type=jax.ShapeDtypeStruct((2, NP, D), jnp.float32),
    mesh=_mesh,
    scratch_types=[
        pltpu.VMEM((G, CHUNK), jnp.int32),
        pltpu.VMEM((G, CHUNK), jnp.int32),
        pltpu.VMEM((G, CHUNK), jnp.int32),
        pltpu.VMEM((G, CHUNK), jnp.int32),
        pltpu.VMEM((CHUNK, D), jnp.float32),
        pltpu.VMEM((CHUNK, D), jnp.float32),
        pltpu.VMEM_SHARED((NP, D), jnp.float32),
        pltpu.SemaphoreType.DMA,
        pltpu.SemaphoreType.DMA,
        pltpu.SemaphoreType.DMA,
        pltpu.SemaphoreType.DMA,
    ],
    compiler_params=_sc_params,
)


def _dinv(cnt_block):
    deg = jnp.sum(cnt_block, axis=0) + 1.0
    return lax.rsqrt(deg)[:, None]


def _tc1_body(x_ref, w_ref, cnt_ref, g_ref):
    h = jnp.dot(x_ref[...], w_ref[...], preferred_element_type=jnp.float32)
    g_ref[...] = h * _dinv(cnt_ref[...])


def _tc2_body(agg_ref, g1_ref, cnt_ref, b_ref, w_ref, g2_ref):
    dinv = _dinv(cnt_ref[...])
    out1 = (agg_ref[0] + agg_ref[1] + g1_ref[...]) * dinv + b_ref[...]
    h2 = jnp.dot(jnp.maximum(out1, 0.0), w_ref[...],
                 preferred_element_type=jnp.float32)
    g2_ref[...] = h2 * dinv


def _tc3_body(agg_ref, g2_ref, cnt_ref, b_ref, out_ref):
    dinv = _dinv(cnt_ref[...])
    out_ref[...] = (agg_ref[0] + agg_ref[1] + g2_ref[...]) * dinv + b_ref[...]


_row_spec = pl.BlockSpec((BR, D), lambda i: (i, 0))
_cnt_spec = pl.BlockSpec((32, BR), lambda i: (0, i))
_full_spec = pl.BlockSpec((D, D), lambda i: (0, 0))
_bias_spec = pl.BlockSpec((1, D), lambda i: (0, 0))
_agg_spec = pl.BlockSpec((2, BR, D), lambda i: (0, i, 0))
_out_sds = jax.ShapeDtypeStruct((NP, D), jnp.float32)

_tc1 = pl.pallas_call(
    _tc1_body, grid=(NP // BR,),
    in_specs=[_row_spec, _full_spec, _cnt_spec],
    out_specs=_row_spec, out_shape=_out_sds)

_tc2 = pl.pallas_call(
    _tc2_body, grid=(NP // BR,),
    in_specs=[_agg_spec, _row_spec, _cnt_spec, _bias_spec, _full_spec],
    out_specs=_row_spec, out_shape=_out_sds)

_tc3 = pl.pallas_call(
    _tc3_body, grid=(NP // BR,),
    in_specs=[_agg_spec, _row_spec, _cnt_spec, _bias_spec],
    out_specs=_row_spec, out_shape=_out_sds)


@jax.jit
def kernel(x, edge_index, W1, b1, W2, b2):
    src = edge_index[0].astype(jnp.int32)
    dst = edge_index[1].astype(jnp.int32)
    pad = jnp.full((EP - E,), N, jnp.int32)
    src_p = jnp.concatenate([src, pad])
    dst_p = jnp.concatenate([dst, pad])
    x_p = jnp.pad(x, ((0, NP - N), (0, 0)))
    zeros = jnp.zeros((NP, D), jnp.float32)
    b1r = b1.reshape(1, D)
    b2r = b2.reshape(1, D)

    src2 = src_p.reshape(EP // CHUNK, CHUNK)
    dst2 = dst_p.reshape(EP // CHUNK, CHUNK)

    counts = _deg_call(dst_p)
    g1 = _tc1(x_p, W1, counts)
    agg1 = _agg_call(g1, src2, dst2, zeros)
    g2 = _tc2(agg1, g1, counts, b1r, W2)
    agg2 = _agg_call(g2, src2, dst2, zeros)
    out = _tc3(agg2, g2, counts, b2r)
    return out[:N]
